# Initial kernel scaffold; baseline (speedup 1.0000x reference)
import jax
import jax.numpy as jnp
from jax.experimental import pallas as pl


def kernel(x, edge_index, W0l, W0r, b0, g0, be0, W1l, W1r, b1, g1, be1, W2l, W2r, b2, m0l, m0r, m1l, m1r, m2l, m2r):
    raise NotImplementedError("write your pallas kernel here")


# speedup vs baseline: 5.7461x; 5.7461x over previous
import functools

import jax
import jax.numpy as jnp
from jax import lax
from jax.experimental import pallas as pl
from jax.experimental.pallas import tpu as pltpu
from jax.experimental.pallas import tpu_sc as plsc

NC = 2
NS = 16
CHUNK = 128


def _n_pad(n_nodes):
    return -(-n_nodes // (NS * CHUNK)) * NS * CHUNK


def _edge_split(n_edges, c, s):
    nchunk = n_edges // CHUNK
    half = nchunk // NC
    base = c * half
    nk = half // NS + jnp.where(s < (half % NS), 1, 0)
    return base, nk


def _sc_agg_body(n_pad, n_edges, dim,
                 tab, ei, zrows, acc_out,
                 acc_sh, src_v, dst_v, rows_v, sem):
    c = lax.axis_index("c")
    s = lax.axis_index("s")
    slab = n_pad // NS
    nstage = slab // CHUNK
    start = s * slab

    pltpu.sync_copy(zrows, rows_v)
    for j in range(nstage):
        pltpu.sync_copy(rows_v, acc_sh.at[pl.ds(start + j * CHUNK, CHUNK)])
    plsc.subcore_barrier()

    base, nk = _edge_split(n_edges, c, s)

    def body(k, carry):
        off = (base + s + NS * k) * CHUNK
        pltpu.sync_copy(ei.at[0, pl.ds(off, CHUNK)], src_v)
        pltpu.sync_copy(ei.at[1, pl.ds(off, CHUNK)], dst_v.at[0])
        pltpu.async_copy(tab.at[src_v], rows_v, sem).wait()
        pltpu.sync_copy(rows_v, acc_sh.at[dst_v.at[0]], add=True)
        return carry

    lax.fori_loop(0, nk, body, 0)
    plsc.subcore_barrier()

    for j in range(nstage):
        pltpu.sync_copy(acc_sh.at[pl.ds(start + j * CHUNK, CHUNK)], rows_v)
        pltpu.sync_copy(rows_v, acc_out.at[c, pl.ds(start + j * CHUNK, CHUNK)])


def _sc_segment_sum(table, edge_index):
    n_nodes, dim = table.shape
    n_edges = edge_index.shape[1]
    n_pad = _n_pad(n_nodes)
    mesh = plsc.VectorSubcoreMesh(core_axis_name="c", subcore_axis_name="s",
                                  num_cores=NC, num_subcores=NS)
    fn = pl.kernel(
        functools.partial(_sc_agg_body, n_pad, n_edges, dim),
        out_type=jax.ShapeDtypeStruct((NC, n_pad, dim), jnp.float32),
        mesh=mesh,
        scratch_types=[
            pltpu.VMEM_SHARED((n_pad, dim), jnp.float32),
            pltpu.VMEM((CHUNK,), jnp.int32),
            pltpu.VMEM((1, CHUNK), jnp.int32),
            pltpu.VMEM((CHUNK, dim), jnp.float32),
            pltpu.SemaphoreType.DMA,
        ],
        name=f"sc_segsum_d{dim}")
    return fn(table, edge_index, jnp.zeros((CHUNK, dim), jnp.float32))


def _sc_deg_body(n_pad, n_edges,
                 ei, zrows, ones_h, cnt_out,
                 cnt_sh, dst_v, rows_v, sem):
    c = lax.axis_index("c")
    s = lax.axis_index("s")
    slab = n_pad // NS
    nstage = slab // CHUNK
    start = s * slab

    pltpu.sync_copy(zrows, rows_v)
    for j in range(nstage):
        pltpu.sync_copy(rows_v, cnt_sh.at[pl.ds(start + j * CHUNK, CHUNK)])
    pltpu.sync_copy(ones_h, rows_v)
    plsc.subcore_barrier()

    base, nk = _edge_split(n_edges, c, s)

    def body(k, carry):
        off = (base + s + NS * k) * CHUNK
        pltpu.sync_copy(ei.at[1, pl.ds(off, CHUNK)], dst_v.at[0])
        pltpu.sync_copy(rows_v, cnt_sh.at[dst_v.at[0]], add=True)
        return carry

    lax.fori_loop(0, nk, body, 0)
    plsc.subcore_barrier()

    for j in range(nstage):
        pltpu.sync_copy(cnt_sh.at[pl.ds(start + j * CHUNK, CHUNK)], rows_v)
        pltpu.sync_copy(rows_v, cnt_out.at[c, pl.ds(start + j * CHUNK, CHUNK)])


def _sc_degree(edge_index, n_nodes):
    n_edges = edge_index.shape[1]
    n_pad = _n_pad(n_nodes)
    dim = 128
    mesh = plsc.VectorSubcoreMesh(core_axis_name="c", subcore_axis_name="s",
                                  num_cores=NC, num_subcores=NS)
    fn = pl.kernel(
        functools.partial(_sc_deg_body, n_pad, n_edges),
        out_type=jax.ShapeDtypeStruct((NC, n_pad, dim), jnp.float32),
        mesh=mesh,
        scratch_types=[
            pltpu.VMEM_SHARED((n_pad, dim), jnp.float32),
            pltpu.VMEM((1, CHUNK), jnp.int32),
            pltpu.VMEM((CHUNK, dim), jnp.float32),
            pltpu.SemaphoreType.DMA,
        ],
        name="sc_degree")
    return fn(edge_index, jnp.zeros((CHUNK, dim), jnp.float32),
              jnp.ones((CHUNK, dim), jnp.float32))


def _mm(a, w):
    return lax.dot_general(a, w, (((1,), (1,)), ((), ())),
                           preferred_element_type=jnp.float32)


def _dense_body(acc_ref, cnt_ref, x_ref, wl_ref, ml_ref, wr_ref, mr_ref,
                b_ref, z_ref, st_ref):
    i = pl.program_id(0)
    summed = acc_ref[0] + acc_ref[1]
    cnt = cnt_ref[0, :, 0:1] + cnt_ref[1, :, 0:1]
    mean = summed / jnp.maximum(cnt, 1.0)
    wl = wl_ref[...] * ml_ref[...]
    wr = wr_ref[...] * mr_ref[...]
    z = _mm(mean, wl) + _mm(x_ref[...], wr) + b_ref[...]
    z_ref[...] = z

    @pl.when(i == 0)
    def _():
        st_ref[...] = jnp.zeros_like(st_ref)

    st_ref[0:1, :] += jnp.sum(z, axis=0, keepdims=True)
    st_ref[1:2, :] += jnp.sum(z * z, axis=0, keepdims=True)


def _dense_layer(acc, cnt, x, wl, ml, wr, mr, b, blk=1000):
    n, d = x.shape
    h = wl.shape[0]
    grid = (n // blk,)
    z, st = pl.pallas_call(
        _dense_body,
        grid=grid,
        in_specs=[
            pl.BlockSpec((NC, blk, acc.shape[2]), lambda i: (0, i, 0)),
            pl.BlockSpec((NC, blk, cnt.shape[2]), lambda i: (0, i, 0)),
            pl.BlockSpec((blk, d), lambda i: (i, 0)),
            pl.BlockSpec((h, d), lambda i: (0, 0)),
            pl.BlockSpec((h, d), lambda i: (0, 0)),
            pl.BlockSpec((h, d), lambda i: (0, 0)),
            pl.BlockSpec((h, d), lambda i: (0, 0)),
            pl.BlockSpec((1, h), lambda i: (0, 0)),
        ],
        out_specs=[
            pl.BlockSpec((blk, h), lambda i: (i, 0)),
            pl.BlockSpec((2, h), lambda i: (0, 0)),
        ],
        out_shape=[
            jax.ShapeDtypeStruct((n, h), jnp.float32),
            jax.ShapeDtypeStruct((2, h), jnp.float32),
        ],
    )(acc, cnt, x, wl, ml, wr, mr, b)
    return z, st


def _bn_coeffs(st, g, be, n, eps=1e-5):
    mu = st[0] / n
    var = st[1] / n - mu * mu
    a = g * lax.rsqrt(var + eps)
    c = be - mu * a
    return a.reshape(1, -1), c.reshape(1, -1)


def _affine_relu_body(z_ref, a_ref, c_ref, h_ref):
    h_ref[...] = jnp.maximum(z_ref[...] * a_ref[...] + c_ref[...], 0.0)


def _affine_relu(z, a, c, blk=1000):
    n, h = z.shape
    return pl.pallas_call(
        _affine_relu_body,
        grid=(n // blk,),
        in_specs=[
            pl.BlockSpec((blk, h), lambda i: (i, 0)),
            pl.BlockSpec((1, h), lambda i: (0, 0)),
            pl.BlockSpec((1, h), lambda i: (0, 0)),
        ],
        out_specs=pl.BlockSpec((blk, h), lambda i: (i, 0)),
        out_shape=jax.ShapeDtypeStruct((n, h), jnp.float32),
    )(z, a, c)


def _affine_relu_mm_body(z_ref, a_ref, c_ref, wl_ref, ml_ref, wr_ref, mr_ref,
                         b_ref, y_ref, r_ref):
    hid = jnp.maximum(z_ref[...] * a_ref[...] + c_ref[...], 0.0)
    y_ref[...] = _mm(hid, wl_ref[...] * ml_ref[...])
    r_ref[...] = _mm(hid, wr_ref[...] * mr_ref[...]) + b_ref[...]


def _affine_relu_mm(z, a, c, wl_pad, ml_pad, wr, mr, b, blk=1000):
    n, h = z.shape
    cp = wl_pad.shape[0]
    co = wr.shape[0]
    return pl.pallas_call(
        _affine_relu_mm_body,
        grid=(n // blk,),
        in_specs=[
            pl.BlockSpec((blk, h), lambda i: (i, 0)),
            pl.BlockSpec((1, h), lambda i: (0, 0)),
            pl.BlockSpec((1, h), lambda i: (0, 0)),
            pl.BlockSpec((cp, h), lambda i: (0, 0)),
            pl.BlockSpec((cp, h), lambda i: (0, 0)),
            pl.BlockSpec((co, h), lambda i: (0, 0)),
            pl.BlockSpec((co, h), lambda i: (0, 0)),
            pl.BlockSpec((1, co), lambda i: (0, 0)),
        ],
        out_specs=[
            pl.BlockSpec((blk, cp), lambda i: (i, 0)),
            pl.BlockSpec((blk, co), lambda i: (i, 0)),
        ],
        out_shape=[
            jax.ShapeDtypeStruct((n, cp), jnp.float32),
            jax.ShapeDtypeStruct((n, co), jnp.float32),
        ],
    )(z, a, c, wl_pad, ml_pad, wr, mr, b)


def _final_body(co, aggy_ref, cnt_ref, r_ref, o_ref):
    cnt = cnt_ref[0, :, 0:1] + cnt_ref[1, :, 0:1]
    my = (aggy_ref[0] + aggy_ref[1])[:, :co] / jnp.maximum(cnt, 1.0)
    logits = my + r_ref[...]
    m = jnp.max(logits, axis=1, keepdims=True)
    lse = jnp.log(jnp.sum(jnp.exp(logits - m), axis=1, keepdims=True)) + m
    o_ref[...] = logits - lse


def _final(aggy, cnt, r, blk=1000):
    n, co = r.shape
    cp = aggy.shape[2]
    return pl.pallas_call(
        functools.partial(_final_body, co),
        grid=(n // blk,),
        in_specs=[
            pl.BlockSpec((NC, blk, cp), lambda i: (0, i, 0)),
            pl.BlockSpec((NC, blk, cnt.shape[2]), lambda i: (0, i, 0)),
            pl.BlockSpec((blk, co), lambda i: (i, 0)),
        ],
        out_specs=pl.BlockSpec((blk, co), lambda i: (i, 0)),
        out_shape=jax.ShapeDtypeStruct((n, co), jnp.float32),
    )(aggy, cnt, r)


def kernel(x, edge_index, W0l, W0r, b0, g0, be0, W1l, W1r, b1, g1, be1,
           W2l, W2r, b2, m0l, m0r, m1l, m1r, m2l, m2r):
    n, d = x.shape
    h = W0l.shape[0]
    co = W2l.shape[0]
    cp = 128

    f32 = jnp.float32
    m0l, m0r = m0l.astype(f32), m0r.astype(f32)
    m1l, m1r = m1l.astype(f32), m1r.astype(f32)
    m2l, m2r = m2l.astype(f32), m2r.astype(f32)
    W2l_pad = jnp.zeros((cp, h), f32).at[:co].set(W2l)
    m2l_pad = jnp.zeros((cp, h), f32).at[:co].set(m2l)

    cnt = _sc_degree(edge_index, n)

    acc0 = _sc_segment_sum(x, edge_index)
    z0, st0 = _dense_layer(acc0, cnt, x, W0l, m0l, W0r, m0r,
                           b0.reshape(1, -1))
    a0, c0 = _bn_coeffs(st0, g0, be0, n)
    h0 = _affine_relu(z0, a0, c0)

    acc1 = _sc_segment_sum(h0, edge_index)
    z1, st1 = _dense_layer(acc1, cnt, h0, W1l, m1l, W1r, m1r,
                           b1.reshape(1, -1))
    a1, c1 = _bn_coeffs(st1, g1, be1, n)

    y, r = _affine_relu_mm(z1, a1, c1, W2l_pad, m2l_pad, W2r, m2r,
                           b2.reshape(1, -1))
    aggy = _sc_segment_sum(y, edge_index)
    return _final(aggy, cnt, r)
